# Initial kernel scaffold; baseline (speedup 1.0000x reference)
import jax
import jax.numpy as jnp
from jax.experimental import pallas as pl


def kernel(x, edge_index, W_in, b_in, Ws, bs, W_out, b_out):
    raise NotImplementedError("write your pallas kernel here")


# speedup vs baseline: 4.5676x; 4.5676x over previous
import functools

import jax
import jax.numpy as jnp
import numpy as np
from jax import lax
from jax.experimental import pallas as pl
from jax.experimental.pallas import tpu as pltpu
from jax.experimental.pallas import tpu_sc as plsc

N = 10000
E = 320000
HID = 128
DIVER = 32

NC = 2
NS = 16
NW = NC * NS
K = 80
CHUNKS = E // (NW * K)
TILE_STRIDE = 624
TILE_CHUNKS = 8


def _make_agg(gather):
    mesh = plsc.VectorSubcoreMesh(core_axis_name="c", subcore_axis_name="s")

    @functools.partial(
        pl.kernel,
        mesh=mesh,
        out_type=jax.ShapeDtypeStruct((NC, N, HID), jnp.float32),
        scratch_types=[
            pltpu.VMEM((K,), jnp.int32),
            pltpu.VMEM((K,), jnp.int32),
            pltpu.VMEM((K, HID), jnp.float32),
            pltpu.VMEM_SHARED((N, HID), jnp.float32),
            pltpu.SemaphoreType.DMA,
        ],
    )
    def agg(g_hbm, src_hbm, dst_hbm, zrows_hbm, out_hbm,
            idx_s, idx_d, rows, acc, sem):
        c = lax.axis_index("c")
        s = lax.axis_index("s")
        wid = s * NC + c
        base = s * TILE_STRIDE

        pltpu.sync_copy(zrows_hbm, rows)
        for t in range(TILE_CHUNKS):
            pltpu.sync_copy(rows, acc.at[pl.ds(base + t * K, K)])
        plsc.subcore_barrier()

        ebase = wid * (CHUNKS * K)
        if not gather:
            pltpu.sync_copy(g_hbm, rows)

        def chunk_body(j, carry):
            eb = ebase + j * K
            pltpu.sync_copy(dst_hbm.at[pl.ds(eb, K)], idx_d)
            if gather:
                pltpu.sync_copy(src_hbm.at[pl.ds(eb, K)], idx_s)
                pltpu.async_copy(g_hbm.at[idx_s], rows, sem).wait()
            pltpu.sync_copy(rows, acc.at[idx_d], add=True)
            return carry

        lax.fori_loop(0, CHUNKS, chunk_body, 0)
        plsc.subcore_barrier()

        for t in range(TILE_CHUNKS):
            pltpu.sync_copy(acc.at[pl.ds(base + t * K, K)], rows)
            pltpu.sync_copy(rows, out_hbm.at[c, pl.ds(base + t * K, K)])

    return agg


_agg128 = _make_agg(True)
_deg_agg = _make_agg(False)


_R = 1000


def _dis_body(d0_ref, d1_ref, out_ref):
    deg = d0_ref[:, :1] + d1_ref[:, :1] + 1.0
    out_ref[...] = lax.rsqrt(deg)


def _dis_tc(d0, d1):
    return pl.pallas_call(
        _dis_body,
        out_shape=jax.ShapeDtypeStruct((N, 1), jnp.float32),
    )(d0, d1)


def _first_body(x_ref, w_ref, b_ref, dis_ref, g_ref):
    z = jnp.dot(x_ref[...], w_ref[...],
                preferred_element_type=jnp.float32) + b_ref[...]
    g_ref[...] = dis_ref[...] * z


def _first_tc(x, w, b, dis):
    din, dout = w.shape
    return pl.pallas_call(
        _first_body,
        grid=(N // _R,),
        in_specs=[
            pl.BlockSpec((_R, din), lambda i: (i, 0)),
            pl.BlockSpec((din, dout), lambda i: (0, 0)),
            pl.BlockSpec((1, dout), lambda i: (0, 0)),
            pl.BlockSpec((_R, 1), lambda i: (i, 0)),
        ],
        out_specs=pl.BlockSpec((_R, dout), lambda i: (i, 0)),
        out_shape=jax.ShapeDtypeStruct((N, dout), jnp.float32),
    )(x, w, b, dis)


def _mid_body(a0_ref, a1_ref, g_ref, w_ref, b_ref, dis_ref, out_ref):
    h = dis_ref[...] * (a0_ref[...] + a1_ref[...] + g_ref[...])
    h = jnp.maximum(h, 0.0)
    z = jnp.dot(h, w_ref[...], preferred_element_type=jnp.float32) + b_ref[...]
    out_ref[...] = dis_ref[...] * z


def _mid_tc(a0, a1, g, w, b, dis):
    din, dout = w.shape
    return pl.pallas_call(
        _mid_body,
        grid=(N // _R,),
        in_specs=[
            pl.BlockSpec((_R, din), lambda i: (i, 0)),
            pl.BlockSpec((_R, din), lambda i: (i, 0)),
            pl.BlockSpec((_R, din), lambda i: (i, 0)),
            pl.BlockSpec((din, dout), lambda i: (0, 0)),
            pl.BlockSpec((1, dout), lambda i: (0, 0)),
            pl.BlockSpec((_R, 1), lambda i: (i, 0)),
        ],
        out_specs=pl.BlockSpec((_R, dout), lambda i: (i, 0)),
        out_shape=jax.ShapeDtypeStruct((N, dout), jnp.float32),
    )(a0, a1, g, w, b, dis)


def _final_body(a0_ref, a1_ref, g_ref, dis_ref, p_ref, out_ref):
    o = dis_ref[...] * (a0_ref[...] + a1_ref[...] + g_ref[...])
    e = jnp.exp(o[:, :DIVER * 2] * 0.01)
    denom = jnp.dot(e, p_ref[...], preferred_element_type=jnp.float32)
    out_ref[...] = e / denom


def _final_tc(a0, a1, g, dis, p):
    dout = DIVER * 2
    return pl.pallas_call(
        _final_body,
        grid=(N // _R,),
        in_specs=[
            pl.BlockSpec((_R, HID), lambda i: (i, 0)),
            pl.BlockSpec((_R, HID), lambda i: (i, 0)),
            pl.BlockSpec((_R, HID), lambda i: (i, 0)),
            pl.BlockSpec((_R, 1), lambda i: (i, 0)),
            pl.BlockSpec((dout, dout), lambda i: (0, 0)),
        ],
        out_specs=pl.BlockSpec((_R, dout), lambda i: (i, 0)),
        out_shape=jax.ShapeDtypeStruct((N, dout), jnp.float32),
    )(a0, a1, g, dis, p)


_PAIR_NP = np.kron(np.eye(DIVER, dtype=np.float32),
                   np.ones((2, 2), dtype=np.float32))


def kernel(x, edge_index, W_in, b_in, Ws, bs, W_out, b_out):
    src = edge_index[0]
    dst = edge_index[1]
    zrows = jnp.zeros((K, HID), jnp.float32)
    orows = jnp.ones((K, HID), jnp.float32)

    degs = _deg_agg(orows, src, dst, zrows)
    dis = _dis_tc(degs[0][:, :8], degs[1][:, :8])

    g = _first_tc(x, W_in, b_in.reshape(1, -1), dis)

    def step(g, wb):
        w, b = wb
        acc = _agg128(g, src, dst, zrows)
        g_new = _mid_tc(acc[0], acc[1], g, w, b.reshape(1, -1), dis)
        return g_new, None

    g, _ = lax.scan(step, g, (Ws, bs))

    w_out_pad = jnp.concatenate(
        [W_out, jnp.zeros((HID, HID - DIVER * 2), jnp.float32)], axis=1)
    b_out_pad = jnp.concatenate(
        [b_out, jnp.zeros((HID - DIVER * 2,), jnp.float32)])

    acc = _agg128(g, src, dst, zrows)
    g_out = _mid_tc(acc[0], acc[1], g, w_out_pad,
                    b_out_pad.reshape(1, -1), dis)

    acc_o = _agg128(g_out, src, dst, zrows)
    return _final_tc(acc_o[0], acc_o[1], g_out, dis,
                     jnp.asarray(_PAIR_NP))
